# baseline (speedup 1.0000x reference)
import functools

import jax
import jax.numpy as jnp
import numpy as np
from jax import lax
from jax.experimental import pallas as pl
from jax.experimental.pallas import tpu as pltpu
from jax.experimental.pallas import tpu_sc as plsc

N = 10000
NP = 10240
E = 320000
CHUNK = 128
NC, NS = 2, 16
NW = NC * NS
E_PAD = 327680
CPT = E_PAD // (NW * CHUNK)
RPT = NP // NS
DEG_W = 16


def _sc_mesh():
    return plsc.VectorSubcoreMesh(
        core_axis_name="c", subcore_axis_name="s", num_cores=NC, num_subcores=NS
    )


GRP = 16
NG = CPT // GRP


def _make_sc_agg(D):
    NRB = 2 if D == 128 else 4

    @functools.partial(
        pl.kernel,
        out_type=jax.ShapeDtypeStruct((NC, NP, D), jnp.float32),
        mesh=_sc_mesh(),
        compiler_params=pltpu.CompilerParams(use_tc_tiling_on_sc=False),
        scratch_types=[
            pltpu.VMEM((GRP, CHUNK), jnp.int32),
            pltpu.VMEM((GRP, CHUNK), jnp.int32),
            pltpu.VMEM((NRB, CHUNK, D), jnp.float32),
            pltpu.VMEM_SHARED((NP, D), jnp.float32),
            pltpu.SemaphoreType.DMA,
            pltpu.SemaphoreType.DMA,
        ],
    )
    def agg_kernel(h_hbm, edges_hbm, zeros_hbm, out_hbm, src_v, dst_v, rows_v, acc, gsem, ssem):
        c = lax.axis_index("c")
        s = lax.axis_index("s")
        w = c * NS + s
        acc_base = pl.multiple_of(s * RPT, CHUNK)
        pltpu.sync_copy(zeros_hbm, acc.at[pl.ds(acc_base, RPT)])
        plsc.subcore_barrier()

        def group(g, _):
            row0 = pl.multiple_of(w * CPT + g * GRP, GRP)
            pltpu.sync_copy(edges_hbm.at[0, pl.ds(row0, GRP), :], src_v)
            pltpu.sync_copy(edges_hbm.at[1, pl.ds(row0, GRP), :], dst_v)
            gat = {}
            sca = {}
            for j in range(NRB - 1):
                gat[j] = pltpu.async_copy(
                    h_hbm.at[src_v.at[j]], rows_v.at[j % NRB], gsem
                )
            for j in range(GRP):
                if j - 1 >= 0:
                    sca[j - 1].wait()
                jn = j + NRB - 1
                if jn < GRP:
                    gat[jn] = pltpu.async_copy(
                        h_hbm.at[src_v.at[jn]], rows_v.at[jn % NRB], gsem
                    )
                gat[j].wait()
                sca[j] = pltpu.async_copy(
                    rows_v.at[j % NRB], acc.at[dst_v.at[j]], ssem, add=True
                )
            sca[GRP - 1].wait()
            return 0

        lax.fori_loop(0, NG, group, 0)
        plsc.subcore_barrier()
        pltpu.sync_copy(acc.at[pl.ds(acc_base, RPT)], out_hbm.at[c, pl.ds(acc_base, RPT)])

    return agg_kernel


_ROWS_BLK = 400
_GRID = N // _ROWS_BLK


def _dis_from_parts(dp_ref):
    deg = dp_ref[0, :, 0:1] + dp_ref[1, :, 0:1] + 1.0
    return lax.rsqrt(deg)


def _tc_pass_a(x, W1, degparts):
    def body(x_ref, w_ref, dp_ref, o_ref):
        dis = _dis_from_parts(dp_ref)
        o_ref[...] = (
            jnp.dot(x_ref[...], w_ref[...], preferred_element_type=jnp.float32) * dis
        )

    return pl.pallas_call(
        body,
        grid=(_GRID,),
        in_specs=[
            pl.BlockSpec((_ROWS_BLK, 128), lambda i: (i, 0)),
            pl.BlockSpec((128, 128), lambda i: (0, 0)),
            pl.BlockSpec((NC, _ROWS_BLK, DEG_W), lambda i: (0, i, 0)),
        ],
        out_specs=pl.BlockSpec((_ROWS_BLK, 128), lambda i: (i, 0)),
        out_shape=jax.ShapeDtypeStruct((N, 128), jnp.float32),
    )(x, W1, degparts)


def _tc_pass_b(s1, h1p, degparts, W2, b1):
    def body(s_ref, h_ref, dp_ref, w_ref, b_ref, o_ref):
        dis = _dis_from_parts(dp_ref)
        z = (s_ref[0] + s_ref[1] + h_ref[...]) * dis + b_ref[...]
        z = jnp.maximum(z, 0.0)
        o_ref[...] = (
            jnp.dot(z, w_ref[...], preferred_element_type=jnp.float32) * dis
        )

    return pl.pallas_call(
        body,
        grid=(_GRID,),
        in_specs=[
            pl.BlockSpec((NC, _ROWS_BLK, 128), lambda i: (0, i, 0)),
            pl.BlockSpec((_ROWS_BLK, 128), lambda i: (i, 0)),
            pl.BlockSpec((NC, _ROWS_BLK, DEG_W), lambda i: (0, i, 0)),
            pl.BlockSpec((128, 64), lambda i: (0, 0)),
            pl.BlockSpec((1, 128), lambda i: (0, 0)),
        ],
        out_specs=pl.BlockSpec((_ROWS_BLK, 64), lambda i: (i, 0)),
        out_shape=jax.ShapeDtypeStruct((N, 64), jnp.float32),
    )(s1, h1p, degparts, W2, b1)


def _tc_pass_c(s2, h2p, degparts, b2):
    def body(s_ref, h_ref, dp_ref, b_ref, o_ref):
        dis = _dis_from_parts(dp_ref)
        o_ref[...] = (s_ref[0] + s_ref[1] + h_ref[...]) * dis + b_ref[...]

    return pl.pallas_call(
        body,
        grid=(_GRID,),
        in_specs=[
            pl.BlockSpec((NC, _ROWS_BLK, 64), lambda i: (0, i, 0)),
            pl.BlockSpec((_ROWS_BLK, 64), lambda i: (i, 0)),
            pl.BlockSpec((NC, _ROWS_BLK, DEG_W), lambda i: (0, i, 0)),
            pl.BlockSpec((1, 64), lambda i: (0, 0)),
        ],
        out_specs=pl.BlockSpec((_ROWS_BLK, 64), lambda i: (i, 0)),
        out_shape=jax.ShapeDtypeStruct((N, 64), jnp.float32),
    )(s2, h2p, degparts, b2)


_PAD_E = E_PAD - E
_PAD3 = np.stack([
    (np.arange(_PAD_E) % N).astype(np.int32).reshape(_PAD_E // CHUNK, CHUNK),
    (N + np.arange(_PAD_E) % (NP - N)).astype(np.int32).reshape(_PAD_E // CHUNK, CHUNK),
])
_ONES_TAB = np.ones((N, DEG_W), np.float32)
_Z16 = np.zeros((RPT, DEG_W), np.float32)
_Z128 = np.zeros((RPT, 128), np.float32)
_Z64 = np.zeros((RPT, 64), np.float32)


def kernel(x, edge_index, cache_name, W1, b1, W2, b2):
    del cache_name
    e = edge_index.astype(jnp.int32)
    edges = jnp.concatenate(
        [e.reshape(2, E // CHUNK, CHUNK), jnp.asarray(_PAD3)], axis=1
    )

    ones_tab = jnp.asarray(_ONES_TAB)
    z16 = jnp.asarray(_Z16)
    z128 = jnp.asarray(_Z128)
    z64 = jnp.asarray(_Z64)

    degparts = _make_sc_agg(DEG_W)(ones_tab, edges, z16)
    h1p = _tc_pass_a(x, W1, degparts)
    s1 = _make_sc_agg(128)(h1p, edges, z128)
    h2p = _tc_pass_b(s1, h1p, degparts, W2, b1.reshape(1, 128))
    s2 = _make_sc_agg(64)(h2p, edges, z64)
    return _tc_pass_c(s2, h2p, degparts, b2.reshape(1, 64))


# speedup vs baseline: 1.0696x; 1.0696x over previous
import functools

import jax
import jax.numpy as jnp
import numpy as np
from jax import lax
from jax.experimental import pallas as pl
from jax.experimental.pallas import tpu as pltpu
from jax.experimental.pallas import tpu_sc as plsc

N = 10000
NP = 10240
E = 320000
CHUNK = 128
NC, NS = 2, 16
NW = NC * NS
E_PAD = 327680
CPT = E_PAD // (NW * CHUNK)
RPT = NP // NS
DEG_W = 8


def _sc_mesh():
    return plsc.VectorSubcoreMesh(
        core_axis_name="c", subcore_axis_name="s", num_cores=NC, num_subcores=NS
    )


GRP = 16
NG = CPT // GRP


def _make_sc_agg(D):
    NRB = 2 if D == 128 else 4

    @functools.partial(
        pl.kernel,
        out_type=jax.ShapeDtypeStruct((NC, NP, D), jnp.float32),
        mesh=_sc_mesh(),
        compiler_params=pltpu.CompilerParams(use_tc_tiling_on_sc=False),
        scratch_types=[
            pltpu.VMEM((GRP, CHUNK), jnp.int32),
            pltpu.VMEM((GRP, CHUNK), jnp.int32),
            pltpu.VMEM((NRB, CHUNK, D), jnp.float32),
            pltpu.VMEM_SHARED((NP, D), jnp.float32),
            pltpu.SemaphoreType.DMA,
            pltpu.SemaphoreType.DMA,
        ],
    )
    def agg_kernel(h_hbm, edges_hbm, zeros_hbm, out_hbm, src_v, dst_v, rows_v, acc, gsem, ssem):
        c = lax.axis_index("c")
        s = lax.axis_index("s")
        w = c * NS + s
        acc_base = pl.multiple_of(s * RPT, CHUNK)
        pltpu.sync_copy(zeros_hbm, acc.at[pl.ds(acc_base, RPT)])
        plsc.subcore_barrier()

        def group(g, _):
            row0 = pl.multiple_of(w * CPT + g * GRP, GRP)
            pltpu.sync_copy(edges_hbm.at[0, pl.ds(row0, GRP), :], src_v)
            pltpu.sync_copy(edges_hbm.at[1, pl.ds(row0, GRP), :], dst_v)
            gat = {}
            sca = {}
            for j in range(NRB - 1):
                gat[j] = pltpu.async_copy(
                    h_hbm.at[src_v.at[j]], rows_v.at[j % NRB], gsem
                )
            for j in range(GRP):
                if j - 1 >= 0:
                    sca[j - 1].wait()
                jn = j + NRB - 1
                if jn < GRP:
                    gat[jn] = pltpu.async_copy(
                        h_hbm.at[src_v.at[jn]], rows_v.at[jn % NRB], gsem
                    )
                gat[j].wait()
                sca[j] = pltpu.async_copy(
                    rows_v.at[j % NRB], acc.at[dst_v.at[j]], ssem, add=True
                )
            sca[GRP - 1].wait()
            return 0

        lax.fori_loop(0, NG, group, 0)
        plsc.subcore_barrier()
        pltpu.sync_copy(acc.at[pl.ds(acc_base, RPT)], out_hbm.at[c, pl.ds(acc_base, RPT)])

    return agg_kernel


_ROWS_BLK = 1000
_GRID = N // _ROWS_BLK


def _dis_from_parts(dp_ref):
    deg = dp_ref[0, :, 0:1] + dp_ref[1, :, 0:1] + 1.0
    return lax.rsqrt(deg)


def _tc_matmul1(x, W1):
    def body(x_ref, w_ref, o_ref):
        o_ref[...] = jnp.dot(
            x_ref[...], w_ref[...], preferred_element_type=jnp.float32
        )

    return pl.pallas_call(
        body,
        grid=(_GRID,),
        in_specs=[
            pl.BlockSpec((_ROWS_BLK, 128), lambda i: (i, 0)),
            pl.BlockSpec((128, 128), lambda i: (0, 0)),
        ],
        out_specs=pl.BlockSpec((_ROWS_BLK, 128), lambda i: (i, 0)),
        out_shape=jax.ShapeDtypeStruct((N, 128), jnp.float32),
    )(x, W1)


def _tc_scale1(u, degparts):
    def body(u_ref, dp_ref, o_ref):
        dis = _dis_from_parts(dp_ref)
        o_ref[...] = u_ref[...] * dis

    return pl.pallas_call(
        body,
        grid=(_GRID,),
        in_specs=[
            pl.BlockSpec((_ROWS_BLK, 128), lambda i: (i, 0)),
            pl.BlockSpec((NC, _ROWS_BLK, DEG_W), lambda i: (0, i, 0)),
        ],
        out_specs=pl.BlockSpec((_ROWS_BLK, 128), lambda i: (i, 0)),
        out_shape=jax.ShapeDtypeStruct((N, 128), jnp.float32),
    )(u, degparts)


def _tc_pass_b(s1, h1p, degparts, W2, b1):
    def body(s_ref, h_ref, dp_ref, w_ref, b_ref, o_ref):
        dis = _dis_from_parts(dp_ref)
        z = (s_ref[0] + s_ref[1] + h_ref[...]) * dis + b_ref[...]
        z = jnp.maximum(z, 0.0)
        o_ref[...] = (
            jnp.dot(z, w_ref[...], preferred_element_type=jnp.float32) * dis
        )

    return pl.pallas_call(
        body,
        grid=(_GRID,),
        in_specs=[
            pl.BlockSpec((NC, _ROWS_BLK, 128), lambda i: (0, i, 0)),
            pl.BlockSpec((_ROWS_BLK, 128), lambda i: (i, 0)),
            pl.BlockSpec((NC, _ROWS_BLK, DEG_W), lambda i: (0, i, 0)),
            pl.BlockSpec((128, 64), lambda i: (0, 0)),
            pl.BlockSpec((1, 128), lambda i: (0, 0)),
        ],
        out_specs=pl.BlockSpec((_ROWS_BLK, 64), lambda i: (i, 0)),
        out_shape=jax.ShapeDtypeStruct((N, 64), jnp.float32),
    )(s1, h1p, degparts, W2, b1)


def _tc_pass_c(s2, h2p, degparts, b2):
    def body(s_ref, h_ref, dp_ref, b_ref, o_ref):
        dis = _dis_from_parts(dp_ref)
        o_ref[...] = (s_ref[0] + s_ref[1] + h_ref[...]) * dis + b_ref[...]

    return pl.pallas_call(
        body,
        grid=(_GRID,),
        in_specs=[
            pl.BlockSpec((NC, _ROWS_BLK, 64), lambda i: (0, i, 0)),
            pl.BlockSpec((_ROWS_BLK, 64), lambda i: (i, 0)),
            pl.BlockSpec((NC, _ROWS_BLK, DEG_W), lambda i: (0, i, 0)),
            pl.BlockSpec((1, 64), lambda i: (0, 0)),
        ],
        out_specs=pl.BlockSpec((_ROWS_BLK, 64), lambda i: (i, 0)),
        out_shape=jax.ShapeDtypeStruct((N, 64), jnp.float32),
    )(s2, h2p, degparts, b2)


_PAD_E = E_PAD - E
_PAD3 = np.stack([
    (np.arange(_PAD_E) % N).astype(np.int32).reshape(_PAD_E // CHUNK, CHUNK),
    (N + np.arange(_PAD_E) % (NP - N)).astype(np.int32).reshape(_PAD_E // CHUNK, CHUNK),
])
_ONES_TAB = np.ones((N, DEG_W), np.float32)
_Z16 = np.zeros((RPT, DEG_W), np.float32)
_Z128 = np.zeros((RPT, 128), np.float32)
_Z64 = np.zeros((RPT, 64), np.float32)


def kernel(x, edge_index, cache_name, W1, b1, W2, b2):
    del cache_name
    e = edge_index.astype(jnp.int32)
    edges = jnp.concatenate(
        [e.reshape(2, E // CHUNK, CHUNK), jnp.asarray(_PAD3)], axis=1
    )

    ones_tab = jnp.asarray(_ONES_TAB)
    z16 = jnp.asarray(_Z16)
    z128 = jnp.asarray(_Z128)
    z64 = jnp.asarray(_Z64)

    degparts = _make_sc_agg(DEG_W)(ones_tab, edges, z16)
    u1 = _tc_matmul1(x, W1)
    h1p = _tc_scale1(u1, degparts)
    s1 = _make_sc_agg(128)(h1p, edges, z128)
    h2p = _tc_pass_b(s1, h1p, degparts, W2, b1.reshape(1, 128))
    s2 = _make_sc_agg(64)(h2p, edges, z64)
    return _tc_pass_c(s2, h2p, degparts, b2.reshape(1, 64))


# speedup vs baseline: 1.1616x; 1.0860x over previous
import functools

import jax
import jax.numpy as jnp
import numpy as np
from jax import lax
from jax.experimental import pallas as pl
from jax.experimental.pallas import tpu as pltpu
from jax.experimental.pallas import tpu_sc as plsc

N = 10000
NP = 10240
E = 320000
CHUNK = 128
NC, NS = 2, 16
NW = NC * NS
E_PAD = 327680
CPT = E_PAD // (NW * CHUNK)
RPT = NP // NS
DEG_W = 8


def _sc_mesh():
    return plsc.VectorSubcoreMesh(
        core_axis_name="c", subcore_axis_name="s", num_cores=NC, num_subcores=NS
    )


GRP = 16
NG = CPT // GRP


def _make_sc_deg():
    NRB = 4

    @functools.partial(
        pl.kernel,
        out_type=jax.ShapeDtypeStruct((NC, NP, DEG_W), jnp.float32),
        mesh=_sc_mesh(),
        compiler_params=pltpu.CompilerParams(use_tc_tiling_on_sc=False),
        scratch_types=[
            pltpu.VMEM((GRP, CHUNK), jnp.int32),
            pltpu.VMEM((NRB, CHUNK, DEG_W), jnp.float32),
            pltpu.VMEM_SHARED((NP, DEG_W), jnp.float32),
            pltpu.SemaphoreType.DMA,
        ],
    )
    def deg_kernel(ones_hbm, edges_hbm, zeros_hbm, out_hbm, dst_v, ones_v, acc, ssem):
        c = lax.axis_index("c")
        s = lax.axis_index("s")
        w = c * NS + s
        acc_base = pl.multiple_of(s * RPT, CHUNK)
        pltpu.sync_copy(zeros_hbm, acc.at[pl.ds(acc_base, RPT)])
        plsc.subcore_barrier()

        def group(g, _):
            row0 = pl.multiple_of(w * CPT + g * GRP, GRP)
            pltpu.sync_copy(edges_hbm.at[1, pl.ds(row0, GRP), :], dst_v)
            pltpu.sync_copy(ones_hbm, ones_v)
            sca = {}
            for j in range(GRP):
                sca[j] = pltpu.async_copy(
                    ones_v.at[j % NRB], acc.at[dst_v.at[j]], ssem, add=True
                )
            for j in range(GRP):
                sca[j].wait()
            return 0

        lax.fori_loop(0, NG, group, 0)
        plsc.subcore_barrier()
        pltpu.sync_copy(acc.at[pl.ds(acc_base, RPT)], out_hbm.at[c, pl.ds(acc_base, RPT)])

    return deg_kernel


def _make_sc_agg(D):
    NRB = 2 if D == 128 else 4

    @functools.partial(
        pl.kernel,
        out_type=jax.ShapeDtypeStruct((NC, NP, D), jnp.float32),
        mesh=_sc_mesh(),
        compiler_params=pltpu.CompilerParams(use_tc_tiling_on_sc=False),
        scratch_types=[
            pltpu.VMEM((GRP, CHUNK), jnp.int32),
            pltpu.VMEM((GRP, CHUNK), jnp.int32),
            pltpu.VMEM((NRB, CHUNK, D), jnp.float32),
            pltpu.VMEM_SHARED((NP, D), jnp.float32),
            pltpu.SemaphoreType.DMA,
            pltpu.SemaphoreType.DMA,
        ],
    )
    def agg_kernel(h_hbm, edges_hbm, zeros_hbm, out_hbm, src_v, dst_v, rows_v, acc, gsem, ssem):
        c = lax.axis_index("c")
        s = lax.axis_index("s")
        w = c * NS + s
        acc_base = pl.multiple_of(s * RPT, CHUNK)
        pltpu.sync_copy(zeros_hbm, acc.at[pl.ds(acc_base, RPT)])
        plsc.subcore_barrier()

        def group(g, _):
            row0 = pl.multiple_of(w * CPT + g * GRP, GRP)
            pltpu.sync_copy(edges_hbm.at[0, pl.ds(row0, GRP), :], src_v)
            pltpu.sync_copy(edges_hbm.at[1, pl.ds(row0, GRP), :], dst_v)
            gat = {}
            sca = {}
            for j in range(NRB - 1):
                gat[j] = pltpu.async_copy(
                    h_hbm.at[src_v.at[j]], rows_v.at[j % NRB], gsem
                )
            for j in range(GRP):
                if j - 1 >= 0:
                    sca[j - 1].wait()
                jn = j + NRB - 1
                if jn < GRP:
                    gat[jn] = pltpu.async_copy(
                        h_hbm.at[src_v.at[jn]], rows_v.at[jn % NRB], gsem
                    )
                gat[j].wait()
                sca[j] = pltpu.async_copy(
                    rows_v.at[j % NRB], acc.at[dst_v.at[j]], ssem, add=True
                )
            sca[GRP - 1].wait()
            return 0

        lax.fori_loop(0, NG, group, 0)
        plsc.subcore_barrier()
        pltpu.sync_copy(acc.at[pl.ds(acc_base, RPT)], out_hbm.at[c, pl.ds(acc_base, RPT)])

    return agg_kernel


_ROWS_BLK = 1000
_GRID = N // _ROWS_BLK


def _dis_from_parts(dp_ref):
    deg = dp_ref[0, :, 0:1] + dp_ref[1, :, 0:1] + 1.0
    return lax.rsqrt(deg)


def _tc_matmul1(x, W1):
    def body(x_ref, w_ref, o_ref):
        o_ref[...] = jnp.dot(
            x_ref[...], w_ref[...], preferred_element_type=jnp.float32
        )

    return pl.pallas_call(
        body,
        grid=(_GRID,),
        in_specs=[
            pl.BlockSpec((_ROWS_BLK, 128), lambda i: (i, 0)),
            pl.BlockSpec((128, 128), lambda i: (0, 0)),
        ],
        out_specs=pl.BlockSpec((_ROWS_BLK, 128), lambda i: (i, 0)),
        out_shape=jax.ShapeDtypeStruct((N, 128), jnp.float32),
    )(x, W1)


def _tc_scale1(u, degparts):
    def body(u_ref, dp_ref, o_ref):
        dis = _dis_from_parts(dp_ref)
        o_ref[...] = u_ref[...] * dis

    return pl.pallas_call(
        body,
        grid=(_GRID,),
        in_specs=[
            pl.BlockSpec((_ROWS_BLK, 128), lambda i: (i, 0)),
            pl.BlockSpec((NC, _ROWS_BLK, DEG_W), lambda i: (0, i, 0)),
        ],
        out_specs=pl.BlockSpec((_ROWS_BLK, 128), lambda i: (i, 0)),
        out_shape=jax.ShapeDtypeStruct((N, 128), jnp.float32),
    )(u, degparts)


def _tc_pass_b(s1, h1p, degparts, W2, b1):
    def body(s_ref, h_ref, dp_ref, w_ref, b_ref, o_ref):
        dis = _dis_from_parts(dp_ref)
        z = (s_ref[0] + s_ref[1] + h_ref[...]) * dis + b_ref[...]
        z = jnp.maximum(z, 0.0)
        o_ref[...] = (
            jnp.dot(z, w_ref[...], preferred_element_type=jnp.float32) * dis
        )

    return pl.pallas_call(
        body,
        grid=(_GRID,),
        in_specs=[
            pl.BlockSpec((NC, _ROWS_BLK, 128), lambda i: (0, i, 0)),
            pl.BlockSpec((_ROWS_BLK, 128), lambda i: (i, 0)),
            pl.BlockSpec((NC, _ROWS_BLK, DEG_W), lambda i: (0, i, 0)),
            pl.BlockSpec((128, 64), lambda i: (0, 0)),
            pl.BlockSpec((1, 128), lambda i: (0, 0)),
        ],
        out_specs=pl.BlockSpec((_ROWS_BLK, 64), lambda i: (i, 0)),
        out_shape=jax.ShapeDtypeStruct((N, 64), jnp.float32),
    )(s1, h1p, degparts, W2, b1)


def _tc_pass_c(s2, h2p, degparts, b2):
    def body(s_ref, h_ref, dp_ref, b_ref, o_ref):
        dis = _dis_from_parts(dp_ref)
        o_ref[...] = (s_ref[0] + s_ref[1] + h_ref[...]) * dis + b_ref[...]

    return pl.pallas_call(
        body,
        grid=(_GRID,),
        in_specs=[
            pl.BlockSpec((NC, _ROWS_BLK, 64), lambda i: (0, i, 0)),
            pl.BlockSpec((_ROWS_BLK, 64), lambda i: (i, 0)),
            pl.BlockSpec((NC, _ROWS_BLK, DEG_W), lambda i: (0, i, 0)),
            pl.BlockSpec((1, 64), lambda i: (0, 0)),
        ],
        out_specs=pl.BlockSpec((_ROWS_BLK, 64), lambda i: (i, 0)),
        out_shape=jax.ShapeDtypeStruct((N, 64), jnp.float32),
    )(s2, h2p, degparts, b2)


_PAD_E = E_PAD - E
_PAD3 = np.stack([
    (np.arange(_PAD_E) % N).astype(np.int32).reshape(_PAD_E // CHUNK, CHUNK),
    (N + np.arange(_PAD_E) % (NP - N)).astype(np.int32).reshape(_PAD_E // CHUNK, CHUNK),
])
_ONES4 = np.ones((4, CHUNK, DEG_W), np.float32)
_Z16 = np.zeros((RPT, DEG_W), np.float32)
_Z128 = np.zeros((RPT, 128), np.float32)
_Z64 = np.zeros((RPT, 64), np.float32)


def kernel(x, edge_index, cache_name, W1, b1, W2, b2):
    del cache_name
    e = edge_index.astype(jnp.int32)
    edges = jnp.concatenate(
        [e.reshape(2, E // CHUNK, CHUNK), jnp.asarray(_PAD3)], axis=1
    )

    ones4 = jnp.asarray(_ONES4)
    z16 = jnp.asarray(_Z16)
    z128 = jnp.asarray(_Z128)
    z64 = jnp.asarray(_Z64)

    degparts = _make_sc_deg()(ones4, edges, z16)
    u1 = _tc_matmul1(x, W1)
    h1p = _tc_scale1(u1, degparts)
    s1 = _make_sc_agg(128)(h1p, edges, z128)
    h2p = _tc_pass_b(s1, h1p, degparts, W2, b1.reshape(1, 128))
    s2 = _make_sc_agg(64)(h2p, edges, z64)
    return _tc_pass_c(s2, h2p, degparts, b2.reshape(1, 64))


# speedup vs baseline: 1.1708x; 1.0079x over previous
import functools

import jax
import jax.numpy as jnp
import numpy as np
from jax import lax
from jax.experimental import pallas as pl
from jax.experimental.pallas import tpu as pltpu
from jax.experimental.pallas import tpu_sc as plsc

N = 10000
NP = 10240
E = 320000
CHUNK = 128
NC, NS = 2, 16
NW = NC * NS
E_PAD = 327680
CPT = E_PAD // (NW * CHUNK)
RPT = NP // NS
DEG_W = 8


def _sc_mesh():
    return plsc.VectorSubcoreMesh(
        core_axis_name="c", subcore_axis_name="s", num_cores=NC, num_subcores=NS
    )


GRP = 16
NG = CPT // GRP


def _make_sc_deg():
    NRB = 4

    @functools.partial(
        pl.kernel,
        out_type=jax.ShapeDtypeStruct((NC, NP, DEG_W), jnp.float32),
        mesh=_sc_mesh(),
        compiler_params=pltpu.CompilerParams(use_tc_tiling_on_sc=False),
        scratch_types=[
            pltpu.VMEM((GRP, CHUNK), jnp.int32),
            pltpu.VMEM((NRB, CHUNK, DEG_W), jnp.float32),
            pltpu.VMEM_SHARED((NP, DEG_W), jnp.float32),
            pltpu.SemaphoreType.DMA,
        ],
    )
    def deg_kernel(ones_hbm, edges_hbm, zeros_hbm, out_hbm, dst_v, ones_v, acc, ssem):
        c = lax.axis_index("c")
        s = lax.axis_index("s")
        w = c * NS + s
        acc_base = pl.multiple_of(s * RPT, CHUNK)
        pltpu.sync_copy(zeros_hbm, acc.at[pl.ds(acc_base, RPT)])
        plsc.subcore_barrier()

        def group(g, _):
            row0 = pl.multiple_of(w * CPT + g * GRP, GRP)
            pltpu.sync_copy(edges_hbm.at[1, pl.ds(row0, GRP), :], dst_v)
            pltpu.sync_copy(ones_hbm, ones_v)
            sca = {}
            for j in range(GRP):
                sca[j] = pltpu.async_copy(
                    ones_v.at[j % NRB], acc.at[dst_v.at[j]], ssem, add=True
                )
            for j in range(GRP):
                sca[j].wait()
            return 0

        lax.fori_loop(0, NG, group, 0)
        plsc.subcore_barrier()
        pltpu.sync_copy(acc.at[pl.ds(acc_base, RPT)], out_hbm.at[c, pl.ds(acc_base, RPT)])

    return deg_kernel


def _make_sc_agg(D):
    NRB = 2 if D == 128 else 4

    @functools.partial(
        pl.kernel,
        out_type=jax.ShapeDtypeStruct((NC, NP, D), jnp.float32),
        mesh=_sc_mesh(),
        compiler_params=pltpu.CompilerParams(use_tc_tiling_on_sc=False),
        scratch_types=[
            pltpu.VMEM((GRP, CHUNK), jnp.int32),
            pltpu.VMEM((GRP, CHUNK), jnp.int32),
            pltpu.VMEM((NRB, CHUNK, D), jnp.float32),
            pltpu.VMEM_SHARED((NP, D), jnp.float32),
            pltpu.SemaphoreType.DMA,
            pltpu.SemaphoreType.DMA,
        ],
    )
    def agg_kernel(h_hbm, edges_hbm, zeros_hbm, out_hbm, src_v, dst_v, rows_v, acc, gsem, ssem):
        c = lax.axis_index("c")
        s = lax.axis_index("s")
        w = c * NS + s
        acc_base = pl.multiple_of(s * RPT, CHUNK)
        pltpu.sync_copy(zeros_hbm, acc.at[pl.ds(acc_base, RPT)])
        plsc.subcore_barrier()

        def group(g, _):
            row0 = pl.multiple_of(w * CPT + g * GRP, GRP)
            pltpu.sync_copy(edges_hbm.at[0, pl.ds(row0, GRP), :], src_v)
            pltpu.sync_copy(edges_hbm.at[1, pl.ds(row0, GRP), :], dst_v)
            gat = {}
            sca = {}
            for j in range(NRB - 1):
                gat[j] = pltpu.async_copy(
                    h_hbm.at[src_v.at[j]], rows_v.at[j % NRB], gsem
                )
            for j in range(GRP):
                if j - 1 >= 0:
                    sca[j - 1].wait()
                jn = j + NRB - 1
                if jn < GRP:
                    gat[jn] = pltpu.async_copy(
                        h_hbm.at[src_v.at[jn]], rows_v.at[jn % NRB], gsem
                    )
                gat[j].wait()
                sca[j] = pltpu.async_copy(
                    rows_v.at[j % NRB], acc.at[dst_v.at[j]], ssem, add=True
                )
            sca[GRP - 1].wait()
            return 0

        lax.fori_loop(0, NG, group, 0)
        plsc.subcore_barrier()
        pltpu.sync_copy(acc.at[pl.ds(acc_base, RPT)], out_hbm.at[c, pl.ds(acc_base, RPT)])

    return agg_kernel


_ROWS_BLK = 2048
_GRID = NP // _ROWS_BLK


def _tc_matmul1(x, W1):
    def body(x_ref, w_ref, o_ref):
        o_ref[...] = jnp.dot(
            x_ref[...], w_ref[...], preferred_element_type=jnp.float32
        )

    return pl.pallas_call(
        body,
        grid=(10,),
        in_specs=[
            pl.BlockSpec((1000, 128), lambda i: (i, 0)),
            pl.BlockSpec((128, 128), lambda i: (0, 0)),
        ],
        out_specs=pl.BlockSpec((1000, 128), lambda i: (i, 0)),
        out_shape=jax.ShapeDtypeStruct((N, 128), jnp.float32),
    )(x, W1)


def _tc_scale1(u, dis):
    def body(u_ref, d_ref, o_ref):
        o_ref[...] = u_ref[...] * d_ref[...]

    return pl.pallas_call(
        body,
        grid=(_GRID,),
        in_specs=[
            pl.BlockSpec((_ROWS_BLK, 128), lambda i: (i, 0)),
            pl.BlockSpec((_ROWS_BLK, 1), lambda i: (i, 0)),
        ],
        out_specs=pl.BlockSpec((_ROWS_BLK, 128), lambda i: (i, 0)),
        out_shape=jax.ShapeDtypeStruct((N, 128), jnp.float32),
    )(u, dis)


def _tc_pass_b(s1, h1p, dis, W2, b1):
    def body(s_ref, h_ref, d_ref, w_ref, b_ref, o_ref):
        dd = d_ref[...]
        z = (s_ref[0] + s_ref[1] + h_ref[...]) * dd + b_ref[...]
        z = jnp.maximum(z, 0.0)
        o_ref[...] = (
            jnp.dot(z, w_ref[...], preferred_element_type=jnp.float32) * dd
        )

    return pl.pallas_call(
        body,
        grid=(_GRID,),
        in_specs=[
            pl.BlockSpec((NC, _ROWS_BLK, 128), lambda i: (0, i, 0)),
            pl.BlockSpec((_ROWS_BLK, 128), lambda i: (i, 0)),
            pl.BlockSpec((_ROWS_BLK, 1), lambda i: (i, 0)),
            pl.BlockSpec((128, 64), lambda i: (0, 0)),
            pl.BlockSpec((1, 128), lambda i: (0, 0)),
        ],
        out_specs=pl.BlockSpec((_ROWS_BLK, 64), lambda i: (i, 0)),
        out_shape=jax.ShapeDtypeStruct((N, 64), jnp.float32),
    )(s1, h1p, dis, W2, b1)


def _tc_pass_c(s2, h2p, dis, b2):
    def body(s_ref, h_ref, d_ref, b_ref, o_ref):
        o_ref[...] = (s_ref[0] + s_ref[1] + h_ref[...]) * d_ref[...] + b_ref[...]

    return pl.pallas_call(
        body,
        grid=(_GRID,),
        in_specs=[
            pl.BlockSpec((NC, _ROWS_BLK, 64), lambda i: (0, i, 0)),
            pl.BlockSpec((_ROWS_BLK, 64), lambda i: (i, 0)),
            pl.BlockSpec((_ROWS_BLK, 1), lambda i: (i, 0)),
            pl.BlockSpec((1, 64), lambda i: (0, 0)),
        ],
        out_specs=pl.BlockSpec((_ROWS_BLK, 64), lambda i: (i, 0)),
        out_shape=jax.ShapeDtypeStruct((N, 64), jnp.float32),
    )(s2, h2p, dis, b2)


_PAD_E = E_PAD - E
_PAD3 = np.stack([
    (np.arange(_PAD_E) % N).astype(np.int32).reshape(_PAD_E // CHUNK, CHUNK),
    (N + np.arange(_PAD_E) % (NP - N)).astype(np.int32).reshape(_PAD_E // CHUNK, CHUNK),
])
_ONES4 = np.ones((4, CHUNK, DEG_W), np.float32)
_Z16 = np.zeros((RPT, DEG_W), np.float32)
_Z128 = np.zeros((RPT, 128), np.float32)
_Z64 = np.zeros((RPT, 64), np.float32)


def kernel(x, edge_index, cache_name, W1, b1, W2, b2):
    del cache_name
    e = edge_index.astype(jnp.int32)
    edges = jnp.concatenate(
        [e.reshape(2, E // CHUNK, CHUNK), jnp.asarray(_PAD3)], axis=1
    )

    ones4 = jnp.asarray(_ONES4)
    z16 = jnp.asarray(_Z16)
    z128 = jnp.asarray(_Z128)
    z64 = jnp.asarray(_Z64)

    degparts = _make_sc_deg()(ones4, edges, z16)
    dis = lax.rsqrt(degparts[0, :, 0] + degparts[1, :, 0] + 1.0)[:, None]
    u1 = _tc_matmul1(x, W1)
    h1p = _tc_scale1(u1, dis)
    s1 = _make_sc_agg(128)(h1p, edges, z128)
    h2p = _tc_pass_b(s1, h1p, dis, W2, b1.reshape(1, 128))
    s2 = _make_sc_agg(64)(h2p, edges, z64)
    return _tc_pass_c(s2, h2p, dis, b2.reshape(1, 64))
